# baseline (speedup 1.0000x reference)
import functools

import jax
import jax.numpy as jnp
from jax import lax
from jax.experimental import pallas as pl
from jax.experimental.pallas import tpu as pltpu
from jax.experimental.pallas import tpu_sc as plsc

NC = 2
NS = 16
L = 16
NW = NC * NS

BATCH = 16384
NF = 32
BPW = BATCH // NW
CHUNK = 128
NCHUNK = BPW // CHUNK
GROUPS_PER_CHUNK = CHUNK // L


def _gmf_body(u_idx_hbm, i_idx_hbm, ut_hbm, it_hbm, par_hbm, out_hbm,
              idx_u, idx_i, rows_u, rows_i, out_v, par_v,
              sems_u, sems_i):
  wid = lax.axis_index("s") * NC + lax.axis_index("c")
  base = pl.multiple_of(wid * BPW, BPW)

  pltpu.sync_copy(u_idx_hbm.at[wid], idx_u)
  pltpu.sync_copy(i_idx_hbm.at[wid], idx_i)
  pltpu.sync_copy(par_hbm, par_v)

  copies = []
  for j in range(NCHUNK):
    cu = pltpu.async_copy(ut_hbm.at[idx_u.at[j]],
                          rows_u.at[pl.ds(j * CHUNK, CHUNK)], sems_u.at[j])
    ci = pltpu.async_copy(it_hbm.at[idx_i.at[j]],
                          rows_i.at[pl.ds(j * CHUNK, CHUNK)], sems_i.at[j])
    copies.append((cu, ci))

  bias = par_v[pl.ds(NF, L)]
  wv0 = par_v[pl.ds(0, L)]
  wv1 = par_v[pl.ds(L, L)]
  w_s = [wv0[k] for k in range(L)] + [wv1[k] for k in range(L)]

  def group_body(g, _):
    rb = pl.multiple_of(g * L, L)
    row_ids = rb + lax.iota(jnp.int32, L)
    acc = jnp.zeros((L,), jnp.float32)
    for f in range(NF):
      col = jnp.full((L,), f, jnp.int32)
      uv = plsc.load_gather(rows_u, [row_ids, col])
      iv = plsc.load_gather(rows_i, [row_ids, col])
      acc = acc + uv * iv * w_s[f]
    x = acc + bias
    res = 5.0 / (1.0 + jnp.exp(-x))
    out_v[pl.ds(rb, L)] = res
    return 0

  for j in range(NCHUNK):
    copies[j][0].wait()
    copies[j][1].wait()
    lax.fori_loop(j * GROUPS_PER_CHUNK, (j + 1) * GROUPS_PER_CHUNK,
                  group_body, 0)

  pltpu.sync_copy(out_v, out_hbm.at[pl.ds(base, BPW)])


@jax.jit
def _gmf(u_idx, i_idx, user_table, item_table, params):
  mesh = plsc.VectorSubcoreMesh(core_axis_name="c", subcore_axis_name="s")
  run = pl.kernel(
      _gmf_body,
      out_type=jax.ShapeDtypeStruct((BATCH,), jnp.float32),
      mesh=mesh,
      compiler_params=pltpu.CompilerParams(needs_layout_passes=False,
                                           use_tc_tiling_on_sc=False),
      scratch_types=[
          pltpu.VMEM((NCHUNK, CHUNK), jnp.int32),
          pltpu.VMEM((NCHUNK, CHUNK), jnp.int32),
          pltpu.VMEM((BPW, NF), jnp.float32),
          pltpu.VMEM((BPW, NF), jnp.float32),
          pltpu.VMEM((BPW,), jnp.float32),
          pltpu.VMEM((NF + L,), jnp.float32),
          pltpu.SemaphoreType.DMA((NCHUNK,)),
          pltpu.SemaphoreType.DMA((NCHUNK,)),
      ],
  )
  return run(u_idx, i_idx, user_table, item_table, params)


def kernel(users, items, user_table, item_table, linear_w, linear_b):
  u_idx = (users - 1).astype(jnp.int32).reshape(NW, NCHUNK, CHUNK)
  i_idx = (items - 1).astype(jnp.int32).reshape(NW, NCHUNK, CHUNK)
  params = jnp.concatenate(
      [linear_w.reshape(-1), jnp.broadcast_to(linear_b, (L,))]
  ).astype(jnp.float32)
  return _gmf(u_idx, i_idx, user_table, item_table, params)
